# baseline (speedup 1.0000x reference)
import functools

import jax
import jax.numpy as jnp
from jax import lax
from jax.experimental import pallas as pl
from jax.experimental.pallas import tpu as pltpu
from jax.experimental.pallas import tpu_sc as plsc

NC = 2
NS = 16
NW = NC * NS
L = 16
E8 = 8
TM = 256


def _router_body(nblocks, E, T, x_ref, wr_ref, br_ref, ids_ref, w_ref,
                 usage_ref, loss_ref):
    i = pl.program_id(0)
    x = x_ref[...]
    wr = wr_ref[...]
    logits = lax.dot_general(
        x, wr, (((1,), (1,)), ((), ())), preferred_element_type=jnp.float32
    ) + br_ref[...]
    m1 = jnp.max(logits, axis=1, keepdims=True)
    eidx = lax.broadcasted_iota(jnp.int32, logits.shape, 1)
    big = jnp.int32(E)
    idx1 = jnp.min(jnp.where(logits == m1, eidx, big), axis=1, keepdims=True)
    neg = jnp.full_like(logits, -jnp.inf)
    l2 = jnp.where(eidx == idx1, neg, logits)
    m2 = jnp.max(l2, axis=1, keepdims=True)
    idx2 = jnp.min(jnp.where(l2 == m2, eidx, big), axis=1, keepdims=True)
    p2 = jnp.exp(m2 - m1)
    wn1 = 1.0 / (1.0 + p2)
    wn2 = p2 * wn1
    ids_ref[...] = jnp.concatenate([idx1, idx2], axis=1)
    w_ref[...] = jnp.concatenate([wn1, wn2], axis=1)

    p = jnp.exp(logits - m1)
    s = p / jnp.sum(p, axis=1, keepdims=True)
    used = (s > 0.0).astype(jnp.float32)
    part = jnp.sum(used, axis=0, keepdims=True)

    @pl.when(i == 0)
    def _():
        usage_ref[...] = jnp.zeros_like(usage_ref)
        loss_ref[...] = jnp.zeros_like(loss_ref)

    usage_ref[...] += part

    @pl.when(i == nblocks - 1)
    def _():
        u = usage_ref[...] / jnp.float32(T)
        tgt = jnp.float32(1.0) / jnp.float32(E)
        loss_ref[...] = jnp.mean((u - tgt) ** 2).reshape(1, 1)


def _run_router(flat, Wr, br, T, E, H):
    TB = 512
    nblocks = T // TB
    return pl.pallas_call(
        functools.partial(_router_body, nblocks, E, T),
        grid=(nblocks,),
        in_specs=[
            pl.BlockSpec((TB, H), lambda i: (i, 0)),
            pl.BlockSpec((E, H), lambda i: (0, 0)),
            pl.BlockSpec((1, E), lambda i: (0, 0)),
        ],
        out_specs=[
            pl.BlockSpec((TB, 2), lambda i: (i, 0)),
            pl.BlockSpec((TB, 2), lambda i: (i, 0)),
            pl.BlockSpec((1, E), lambda i: (0, 0)),
            pl.BlockSpec((1, 1), lambda i: (0, 0)),
        ],
        out_shape=[
            jax.ShapeDtypeStruct((T, 2), jnp.int32),
            jax.ShapeDtypeStruct((T, 2), jnp.float32),
            jax.ShapeDtypeStruct((1, E), jnp.float32),
            jax.ShapeDtypeStruct((1, 1), jnp.float32),
        ],
    )(flat, Wr, br.reshape(1, E))


def _disp1_body(AW, ids_hbm, counts_hbm, rank_hbm, ids_v, rank_v, cbuf_v):
    wid = lax.axis_index("s") * NC + lax.axis_index("c")
    base = wid * AW
    pltpu.sync_copy(ids_hbm.at[pl.ds(base, AW)], ids_v)
    cnt = [jnp.zeros((L,), jnp.int32) for _ in range(E8)]
    for q in range(AW // L):
        v = ids_v[pl.ds(q * L, L)]
        r = jnp.zeros((L,), jnp.int32)
        for e in range(E8):
            m = v == e
            r = jnp.where(m, cnt[e], r)
            cnt[e] = cnt[e] + jnp.where(m, 1, 0)
        rank_v[pl.ds(q * L, L)] = r
    for e in range(E8):
        cbuf_v[pl.ds(e * L, L)] = cnt[e]
    pltpu.sync_copy(rank_v, rank_hbm.at[pl.ds(base, AW)])
    pltpu.sync_copy(cbuf_v, counts_hbm.at[wid])


def _run_disp1(ids_t, T2):
    AW = T2 // NW
    return pl.kernel(
        functools.partial(_disp1_body, AW),
        out_type=[
            jax.ShapeDtypeStruct((NW, E8 * L), jnp.int32),
            jax.ShapeDtypeStruct((T2,), jnp.int32),
        ],
        mesh=plsc.VectorSubcoreMesh(core_axis_name="c", subcore_axis_name="s"),
        scratch_types=[
            pltpu.VMEM((AW,), jnp.int32),
            pltpu.VMEM((AW,), jnp.int32),
            pltpu.VMEM((E8 * L,), jnp.int32),
        ],
    )(ids_t)


def _prefix_body(NTPAD, cnt_ref, base_ref, grp_ref):
    EL = E8 * L
    cnt = cnt_ref[...].astype(jnp.float32)
    f32 = jnp.float32

    def dotf(a, b, dims):
        return lax.dot_general(a, b, (dims, ((), ())),
                               preferred_element_type=f32)

    wi = lax.broadcasted_iota(jnp.int32, (NW, NW), 0)
    wj = lax.broadcasted_iota(jnp.int32, (NW, NW), 1)
    strict_w = jnp.where(wj < wi, f32(1.0), f32(0.0))
    colsum = dotf(strict_w, cnt, ((1,), (0,)))

    ones_col = jnp.ones((NW, 1), f32)
    csum_col = dotf(cnt, ones_col, ((0,), (0,)))

    gi = lax.broadcasted_iota(jnp.int32, (EL, EL), 0)
    gj = lax.broadcasted_iota(jnp.int32, (EL, EL), 1)
    same_grp = (gi // L) == (gj // L)
    m_lane = jnp.where(same_grp & ((gi % L) < (gj % L)), f32(1.0), f32(0.0))
    lane_pre = dotf(csum_col, m_lane, ((0,), (0,)))

    ge8 = lax.broadcasted_iota(jnp.int32, (EL, E8), 1)
    gsel = jnp.where((gi[:, :E8] // L) == ge8, f32(1.0), f32(0.0))
    etot_col = dotf(gsel, csum_col, ((0,), (0,)))
    padded = jnp.floor((etot_col + (TM - 1)) * f32(1.0 / TM)) * TM

    ei = lax.broadcasted_iota(jnp.int32, (E8, E8), 0)
    ej = lax.broadcasted_iota(jnp.int32, (E8, E8), 1)
    strict_e = jnp.where(ei < ej, f32(1.0), f32(0.0))
    ebase_col = dotf(strict_e, padded, ((0,), (0,)))

    gsel_t = jnp.where(ge8 == (gi[:, :E8] // L), f32(1.0), f32(0.0))
    ebase_row = dotf(ebase_col, gsel_t, ((0,), (1,)))

    base_ref[...] = (colsum + ebase_row + lane_pre).astype(jnp.int32)

    tstart = (ebase_col * f32(1.0 / TM)).astype(jnp.int32)
    tid = lax.broadcasted_iota(jnp.int32, (E8, NTPAD), 1)
    ge = jnp.where(tid >= tstart, f32(1.0), f32(0.0))
    gids = (jnp.sum(ge, axis=0, keepdims=True) - 1.0).astype(jnp.int32)
    ntiles_used = (jnp.sum(padded) * f32(1.0 / TM)).astype(jnp.int32)
    valid = jnp.where(tid[:1] < ntiles_used, 1, 0)
    grp_ref[...] = jnp.concatenate([gids, valid], axis=0)


def _run_prefix(counts, NTPAD):
    return pl.pallas_call(
        functools.partial(_prefix_body, NTPAD),
        grid=(1,),
        in_specs=[pl.BlockSpec((NW, E8 * L), lambda i: (0, 0))],
        out_specs=[
            pl.BlockSpec((NW, E8 * L), lambda i: (0, 0)),
            pl.BlockSpec((2, NTPAD), lambda i: (0, 0)),
        ],
        out_shape=[
            jax.ShapeDtypeStruct((NW, E8 * L), jnp.int32),
            jax.ShapeDtypeStruct((2, NTPAD), jnp.int32),
        ],
    )(counts)


def _disp2_body(AW, ids_hbm, w_hbm, rank_hbm, base_hbm,
                pos_hbm, rowtok_hbm, roww_hbm,
                ids_v, rank_v, base_v, pos_v, pos2_v, tok2_v, w2_v, sem):
    wid = lax.axis_index("s") * NC + lax.axis_index("c")
    base = wid * AW
    lane = lax.iota(jnp.int32, L)
    pltpu.sync_copy(ids_hbm.at[pl.ds(base, AW)], ids_v)
    pltpu.sync_copy(rank_hbm.at[pl.ds(base, AW)], rank_v)
    pltpu.sync_copy(base_hbm.at[pl.ds(wid * (E8 * L), E8 * L)], base_v)
    for q in range(AW // L):
        v = ids_v[pl.ds(q * L, L)]
        r = rank_v[pl.ds(q * L, L)]
        p = r
        for e in range(E8):
            p = p + jnp.where(v == e, base_v[pl.ds(e * L, L)], 0)
        pos_v[pl.ds(q * L, L)] = p
        row = q // (128 // L)
        col = (q % (128 // L)) * L
        pos2_v[row, pl.ds(col, L)] = p
        av = jnp.full((L,), base + q, jnp.int32) + lane * L
        tok2_v[row, pl.ds(col, L)] = lax.shift_right_logical(av, 1)
    pltpu.sync_copy(pos_v, pos_hbm.at[pl.ds(base, AW)])
    pltpu.sync_copy(w_hbm.at[pl.ds(base, 128)], w2_v.at[0])
    pltpu.sync_copy(w_hbm.at[pl.ds(base + 128, 128)], w2_v.at[1])
    c0 = pltpu.async_copy(tok2_v.at[0], rowtok_hbm.at[pos2_v.at[0]], sem)
    c1 = pltpu.async_copy(tok2_v.at[1], rowtok_hbm.at[pos2_v.at[1]], sem)
    c2 = pltpu.async_copy(w2_v.at[0], roww_hbm.at[pos2_v.at[0]], sem)
    c3 = pltpu.async_copy(w2_v.at[1], roww_hbm.at[pos2_v.at[1]], sem)
    c0.wait()
    c1.wait()
    c2.wait()
    c3.wait()


def _run_disp2(ids_t, w_t, rank_t, base_sw, T2, P):
    AW = T2 // NW
    return pl.kernel(
        functools.partial(_disp2_body, AW),
        out_type=[
            jax.ShapeDtypeStruct((T2,), jnp.int32),
            jax.ShapeDtypeStruct((P,), jnp.int32),
            jax.ShapeDtypeStruct((P,), jnp.float32),
        ],
        mesh=plsc.VectorSubcoreMesh(core_axis_name="c", subcore_axis_name="s"),
        scratch_types=[
            pltpu.VMEM((AW,), jnp.int32),
            pltpu.VMEM((AW,), jnp.int32),
            pltpu.VMEM((E8 * L,), jnp.int32),
            pltpu.VMEM((AW,), jnp.int32),
            pltpu.VMEM((2, 128), jnp.int32),
            pltpu.VMEM((2, 128), jnp.int32),
            pltpu.VMEM((2, 128), jnp.float32),
            pltpu.SemaphoreType.DMA,
        ],
    )(ids_t, w_t, rank_t, base_sw.reshape(NW * E8 * L))


def _gather_body(RW, CH, T, rowtok_hbm, x_hbm, xs_hbm, idx_v, buf_v, sem):
    wid = lax.axis_index("s") * NC + lax.axis_index("c")
    base = wid * RW
    nch = RW // CH
    for c in range(nch):
        pltpu.sync_copy(rowtok_hbm.at[pl.ds(base + c * CH, CH)], idx_v.at[c])
    mask = jnp.full((L,), T - 1, jnp.int32)
    for c in range(nch):
        for q in range(CH // L):
            idx_v[c, pl.ds(q * L, L)] = idx_v[c, pl.ds(q * L, L)] & mask
    pltpu.async_copy(x_hbm.at[idx_v.at[0]], buf_v.at[0], sem)
    if nch > 1:
        pltpu.async_copy(x_hbm.at[idx_v.at[1]], buf_v.at[1], sem)
    for c in range(nch):
        pltpu.make_async_copy(
            x_hbm.at[idx_v.at[c]], buf_v.at[c % 2], sem).wait()
        pltpu.sync_copy(buf_v.at[c % 2], xs_hbm.at[pl.ds(base + c * CH, CH)])
        if c + 2 < nch:
            pltpu.async_copy(x_hbm.at[idx_v.at[c + 2]], buf_v.at[c % 2], sem)


def _run_gather(rowtok, flat, P, T, H):
    RW = P // NW
    CH = 32
    return pl.kernel(
        functools.partial(_gather_body, RW, CH, T),
        out_type=jax.ShapeDtypeStruct((P, H), jnp.float32),
        mesh=plsc.VectorSubcoreMesh(core_axis_name="c", subcore_axis_name="s"),
        scratch_types=[
            pltpu.VMEM((RW // CH, CH), jnp.int32),
            pltpu.VMEM((2, CH, H), jnp.float32),
            pltpu.SemaphoreType.DMA,
        ],
    )(rowtok, flat)


def _gmm1_body(g_ref, x_ref, w1_ref, b1_ref, h_ref):
    t = pl.program_id(0)

    @pl.when(g_ref[1, t] == 1)
    def _():
        x = x_ref[...].astype(jnp.bfloat16)
        h = lax.dot_general(
            x, w1_ref[0], (((1,), (1,)), ((), ())),
            preferred_element_type=jnp.float32) + b1_ref[0]
        h = 0.5 * h * (1.0 + lax.erf(h * jnp.float32(0.7071067811865476)))
        h_ref[...] = h.astype(jnp.bfloat16)


def _run_gmm1(group, xs, W1b, b1, P, E, F, H):
    NT = P // TM
    grid_spec = pltpu.PrefetchScalarGridSpec(
        num_scalar_prefetch=1,
        grid=(NT,),
        in_specs=[
            pl.BlockSpec((TM, H), lambda t, g: (t, 0)),
            pl.BlockSpec((1, F, H), lambda t, g: (g[0, t], 0, 0)),
            pl.BlockSpec((1, 1, F), lambda t, g: (g[0, t], 0, 0)),
        ],
        out_specs=pl.BlockSpec((TM, F), lambda t, g: (t, 0)),
    )
    return pl.pallas_call(
        _gmm1_body,
        grid_spec=grid_spec,
        out_shape=jax.ShapeDtypeStruct((P, F), jnp.bfloat16),
    )(group, xs, W1b, b1.reshape(E, 1, F))


def _gmm2_body(g_ref, h_ref, w2_ref, b2_ref, w_ref, y_ref):
    t = pl.program_id(0)

    @pl.when(g_ref[1, t] == 1)
    def _():
        h = h_ref[...]
        y = lax.dot_general(
            h, w2_ref[0], (((1,), (1,)), ((), ())),
            preferred_element_type=jnp.float32) + b2_ref[0]
        y_ref[...] = y * w_ref[...]


def _run_gmm2(group, h, W2b, b2, roww, P, E, F, V):
    NT = P // TM
    grid_spec = pltpu.PrefetchScalarGridSpec(
        num_scalar_prefetch=1,
        grid=(NT,),
        in_specs=[
            pl.BlockSpec((TM, F), lambda t, g: (t, 0)),
            pl.BlockSpec((1, V, F), lambda t, g: (g[0, t], 0, 0)),
            pl.BlockSpec((1, 1, V), lambda t, g: (g[0, t], 0, 0)),
            pl.BlockSpec((TM, 1), lambda t, g: (t, 0)),
        ],
        out_specs=pl.BlockSpec((TM, V), lambda t, g: (t, 0)),
    )
    return pl.pallas_call(
        _gmm2_body,
        grid_spec=grid_spec,
        out_shape=jax.ShapeDtypeStruct((P, V), jnp.float32),
    )(group, h, W2b, b2.reshape(E, 1, V), roww)


def _combine_body(TW, V, y_hbm, pos_hbm, out_hbm, pos_v, ybuf_v, obuf_v, sem):
    wid = lax.axis_index("s") * NC + lax.axis_index("c")
    tbase = wid * TW
    ngrp = (2 * TW) // L
    for g in range(ngrp):
        pltpu.sync_copy(pos_hbm.at[pl.ds(2 * tbase + g * L, L)], pos_v.at[g])

    def body(g, _):
        pltpu.async_copy(y_hbm.at[pos_v.at[g]], ybuf_v.at[0], sem).wait()

        def inner(q, _):
            for i in range(L // 2):
                obuf_v[i, pl.ds(q * L, L)] = (
                    ybuf_v[0, 2 * i, pl.ds(q * L, L)]
                    + ybuf_v[0, 2 * i + 1, pl.ds(q * L, L)]
                )
            return 0

        lax.fori_loop(0, V // L, inner, 0)
        pltpu.sync_copy(obuf_v, out_hbm.at[pl.ds(tbase + g * (L // 2), L // 2)])
        return 0

    lax.fori_loop(0, ngrp, body, 0)


def _run_combine(y, pos, T, V):
    TW = T // NW
    return pl.kernel(
        functools.partial(_combine_body, TW, V),
        out_type=jax.ShapeDtypeStruct((T, V), jnp.float32),
        mesh=plsc.VectorSubcoreMesh(core_axis_name="c", subcore_axis_name="s"),
        scratch_types=[
            pltpu.VMEM(((2 * TW) // L, L), jnp.int32),
            pltpu.VMEM((2, L, V), jnp.float32),
            pltpu.VMEM((L // 2, V), jnp.float32),
            pltpu.SemaphoreType.DMA,
        ],
    )(y, pos)


def kernel(hidden_states, Wr, br, W1, b1, W2, b2):
    B, S, H = hidden_states.shape
    E, F, _ = W1.shape
    V = W2.shape[1]
    T = B * S
    T2 = 2 * T
    NT = T2 // TM + E
    NTPAD = ((NT + 15) // 16) * 16
    P = NT * TM

    flat = hidden_states.reshape(T, H)
    w1b = W1.astype(jnp.bfloat16)
    w2b = W2.astype(jnp.bfloat16)

    ids2, w2, _usage, loss11 = _run_router(flat, Wr, br, T, E, H)
    ids_t = ids2.reshape(NW, L, L).transpose(0, 2, 1).reshape(T2)
    w_t = w2.reshape(NW, L, L).transpose(0, 2, 1).reshape(T2)

    counts, rank_t = _run_disp1(ids_t, T2)
    base_sw, group2 = _run_prefix(counts, NTPAD)
    pos_t, rowtok, roww = _run_disp2(ids_t, w_t, rank_t, base_sw, T2, P)
    pos = pos_t.reshape(NW, L, L).transpose(0, 2, 1).reshape(T2)
    xs = _run_gather(rowtok, flat, P, T, H)
    group = group2[:, :NT]
    h = _run_gmm1(group, xs, w1b, b1, P, E, F, H)
    y = _run_gmm2(group, h, w2b, b2, roww.reshape(P, 1), P, E, F, V)
    out = _run_combine(y, pos, T, V)

    return out.reshape(B, S, V), loss11[0, 0]


# speedup vs baseline: 1.1388x; 1.1388x over previous
import functools

import jax
import jax.numpy as jnp
from jax import lax
from jax.experimental import pallas as pl
from jax.experimental.pallas import tpu as pltpu
from jax.experimental.pallas import tpu_sc as plsc

NC = 2
NS = 16
NW = NC * NS
L = 16
E8 = 8
TM = 256


def _router_body(nblocks, E, T, x_ref, wr_ref, br_ref, ids_ref, w_ref,
                 usage_ref, loss_ref):
    i = pl.program_id(0)
    x = x_ref[...]
    wr = wr_ref[...]
    logits = lax.dot_general(
        x, wr, (((1,), (1,)), ((), ())), preferred_element_type=jnp.float32
    ) + br_ref[...]
    m1 = jnp.max(logits, axis=1, keepdims=True)
    eidx = lax.broadcasted_iota(jnp.int32, logits.shape, 1)
    big = jnp.int32(E)
    idx1 = jnp.min(jnp.where(logits == m1, eidx, big), axis=1, keepdims=True)
    neg = jnp.full_like(logits, -jnp.inf)
    l2 = jnp.where(eidx == idx1, neg, logits)
    m2 = jnp.max(l2, axis=1, keepdims=True)
    idx2 = jnp.min(jnp.where(l2 == m2, eidx, big), axis=1, keepdims=True)
    p2 = jnp.exp(m2 - m1)
    wn1 = 1.0 / (1.0 + p2)
    wn2 = p2 * wn1
    ids_ref[...] = jnp.concatenate([idx1, idx2], axis=1)
    w_ref[...] = jnp.concatenate([wn1, wn2], axis=1)

    p = jnp.exp(logits - m1)
    s = p / jnp.sum(p, axis=1, keepdims=True)
    used = (s > 0.0).astype(jnp.float32)
    part = jnp.sum(used, axis=0, keepdims=True)

    @pl.when(i == 0)
    def _():
        usage_ref[...] = jnp.zeros_like(usage_ref)
        loss_ref[...] = jnp.zeros_like(loss_ref)

    usage_ref[...] += part

    @pl.when(i == nblocks - 1)
    def _():
        u = usage_ref[...] / jnp.float32(T)
        tgt = jnp.float32(1.0) / jnp.float32(E)
        loss_ref[...] = jnp.mean((u - tgt) ** 2).reshape(1, 1)


def _run_router(flat, Wr, br, T, E, H):
    TB = 512
    nblocks = T // TB
    return pl.pallas_call(
        functools.partial(_router_body, nblocks, E, T),
        grid=(nblocks,),
        in_specs=[
            pl.BlockSpec((TB, H), lambda i: (i, 0)),
            pl.BlockSpec((E, H), lambda i: (0, 0)),
            pl.BlockSpec((1, E), lambda i: (0, 0)),
        ],
        out_specs=[
            pl.BlockSpec((TB, 2), lambda i: (i, 0)),
            pl.BlockSpec((TB, 2), lambda i: (i, 0)),
            pl.BlockSpec((1, E), lambda i: (0, 0)),
            pl.BlockSpec((1, 1), lambda i: (0, 0)),
        ],
        out_shape=[
            jax.ShapeDtypeStruct((T, 2), jnp.int32),
            jax.ShapeDtypeStruct((T, 2), jnp.float32),
            jax.ShapeDtypeStruct((1, E), jnp.float32),
            jax.ShapeDtypeStruct((1, 1), jnp.float32),
        ],
    )(flat, Wr, br.reshape(1, E))


def _disp1_body(AW, ids_hbm, counts_hbm, rank_hbm, ids_v, rank_v, cbuf_v):
    wid = lax.axis_index("s") * NC + lax.axis_index("c")
    base = wid * AW
    pltpu.sync_copy(ids_hbm.at[pl.ds(base, AW)], ids_v)
    cnt = [jnp.zeros((L,), jnp.int32) for _ in range(E8)]
    for q in range(AW // L):
        v = ids_v[pl.ds(q * L, L)]
        r = jnp.zeros((L,), jnp.int32)
        for e in range(E8):
            m = v == e
            r = jnp.where(m, cnt[e], r)
            cnt[e] = cnt[e] + jnp.where(m, 1, 0)
        rank_v[pl.ds(q * L, L)] = r
    for e in range(E8):
        cbuf_v[pl.ds(e * L, L)] = cnt[e]
    pltpu.sync_copy(rank_v, rank_hbm.at[pl.ds(base, AW)])
    pltpu.sync_copy(cbuf_v, counts_hbm.at[wid])


def _run_disp1(ids_t, T2):
    AW = T2 // NW
    return pl.kernel(
        functools.partial(_disp1_body, AW),
        out_type=[
            jax.ShapeDtypeStruct((NW, E8 * L), jnp.int32),
            jax.ShapeDtypeStruct((T2,), jnp.int32),
        ],
        mesh=plsc.VectorSubcoreMesh(core_axis_name="c", subcore_axis_name="s"),
        scratch_types=[
            pltpu.VMEM((AW,), jnp.int32),
            pltpu.VMEM((AW,), jnp.int32),
            pltpu.VMEM((E8 * L,), jnp.int32),
        ],
    )(ids_t)


def _prefix_body(NTPAD, cnt_ref, base_ref, grp_ref):
    EL = E8 * L
    cnt = cnt_ref[...].astype(jnp.float32)
    f32 = jnp.float32

    def dotf(a, b, dims):
        return lax.dot_general(a, b, (dims, ((), ())),
                               preferred_element_type=f32)

    wi = lax.broadcasted_iota(jnp.int32, (NW, NW), 0)
    wj = lax.broadcasted_iota(jnp.int32, (NW, NW), 1)
    strict_w = jnp.where(wj < wi, f32(1.0), f32(0.0))
    colsum = dotf(strict_w, cnt, ((1,), (0,)))

    ones_col = jnp.ones((NW, 1), f32)
    csum_col = dotf(cnt, ones_col, ((0,), (0,)))

    gi = lax.broadcasted_iota(jnp.int32, (EL, EL), 0)
    gj = lax.broadcasted_iota(jnp.int32, (EL, EL), 1)
    same_grp = (gi // L) == (gj // L)
    m_lane = jnp.where(same_grp & ((gi % L) < (gj % L)), f32(1.0), f32(0.0))
    lane_pre = dotf(csum_col, m_lane, ((0,), (0,)))

    ge8 = lax.broadcasted_iota(jnp.int32, (EL, E8), 1)
    gsel = jnp.where((gi[:, :E8] // L) == ge8, f32(1.0), f32(0.0))
    etot_col = dotf(gsel, csum_col, ((0,), (0,)))
    padded = jnp.floor((etot_col + (TM - 1)) * f32(1.0 / TM)) * TM

    ei = lax.broadcasted_iota(jnp.int32, (E8, E8), 0)
    ej = lax.broadcasted_iota(jnp.int32, (E8, E8), 1)
    strict_e = jnp.where(ei < ej, f32(1.0), f32(0.0))
    ebase_col = dotf(strict_e, padded, ((0,), (0,)))

    gsel_t = jnp.where(ge8 == (gi[:, :E8] // L), f32(1.0), f32(0.0))
    ebase_row = dotf(ebase_col, gsel_t, ((0,), (1,)))

    base_ref[...] = (colsum + ebase_row + lane_pre).astype(jnp.int32)

    tstart = (ebase_col * f32(1.0 / TM)).astype(jnp.int32)
    tid = lax.broadcasted_iota(jnp.int32, (E8, NTPAD), 1)
    ge = jnp.where(tid >= tstart, f32(1.0), f32(0.0))
    gids = (jnp.sum(ge, axis=0, keepdims=True) - 1.0).astype(jnp.int32)
    ntiles_used = (jnp.sum(padded) * f32(1.0 / TM)).astype(jnp.int32)
    valid = jnp.where(tid[:1] < ntiles_used, 1, 0)
    grp_ref[...] = jnp.concatenate([gids, valid], axis=0)


def _run_prefix(counts, NTPAD):
    return pl.pallas_call(
        functools.partial(_prefix_body, NTPAD),
        grid=(1,),
        in_specs=[pl.BlockSpec((NW, E8 * L), lambda i: (0, 0))],
        out_specs=[
            pl.BlockSpec((NW, E8 * L), lambda i: (0, 0)),
            pl.BlockSpec((2, NTPAD), lambda i: (0, 0)),
        ],
        out_shape=[
            jax.ShapeDtypeStruct((NW, E8 * L), jnp.int32),
            jax.ShapeDtypeStruct((2, NTPAD), jnp.int32),
        ],
    )(counts)


def _disp2_body(AW, ids_hbm, w_hbm, rank_hbm, base_hbm,
                pos_hbm, rowtok_hbm, roww_hbm,
                ids_v, rank_v, base_v, pos_v, pos2_v, tok2_v, w2_v, sem):
    wid = lax.axis_index("s") * NC + lax.axis_index("c")
    base = wid * AW
    lane = lax.iota(jnp.int32, L)
    pltpu.sync_copy(ids_hbm.at[pl.ds(base, AW)], ids_v)
    pltpu.sync_copy(rank_hbm.at[pl.ds(base, AW)], rank_v)
    pltpu.sync_copy(base_hbm.at[pl.ds(wid * (E8 * L), E8 * L)], base_v)
    for q in range(AW // L):
        v = ids_v[pl.ds(q * L, L)]
        r = rank_v[pl.ds(q * L, L)]
        p = r
        for e in range(E8):
            p = p + jnp.where(v == e, base_v[pl.ds(e * L, L)], 0)
        pos_v[pl.ds(q * L, L)] = p
        row = q // (128 // L)
        col = (q % (128 // L)) * L
        pos2_v[row, pl.ds(col, L)] = p
        av = jnp.full((L,), base + q, jnp.int32) + lane * L
        tok2_v[row, pl.ds(col, L)] = lax.shift_right_logical(av, 1)
    pltpu.sync_copy(pos_v, pos_hbm.at[pl.ds(base, AW)])
    pltpu.sync_copy(w_hbm.at[pl.ds(base, 128)], w2_v.at[0])
    pltpu.sync_copy(w_hbm.at[pl.ds(base + 128, 128)], w2_v.at[1])
    c0 = pltpu.async_copy(tok2_v.at[0], rowtok_hbm.at[pos2_v.at[0]], sem)
    c1 = pltpu.async_copy(tok2_v.at[1], rowtok_hbm.at[pos2_v.at[1]], sem)
    c2 = pltpu.async_copy(w2_v.at[0], roww_hbm.at[pos2_v.at[0]], sem)
    c3 = pltpu.async_copy(w2_v.at[1], roww_hbm.at[pos2_v.at[1]], sem)
    c0.wait()
    c1.wait()
    c2.wait()
    c3.wait()


def _run_disp2(ids_t, w_t, rank_t, base_sw, T2, P):
    AW = T2 // NW
    return pl.kernel(
        functools.partial(_disp2_body, AW),
        out_type=[
            jax.ShapeDtypeStruct((T2,), jnp.int32),
            jax.ShapeDtypeStruct((P,), jnp.int32),
            jax.ShapeDtypeStruct((P,), jnp.float32),
        ],
        mesh=plsc.VectorSubcoreMesh(core_axis_name="c", subcore_axis_name="s"),
        scratch_types=[
            pltpu.VMEM((AW,), jnp.int32),
            pltpu.VMEM((AW,), jnp.int32),
            pltpu.VMEM((E8 * L,), jnp.int32),
            pltpu.VMEM((AW,), jnp.int32),
            pltpu.VMEM((2, 128), jnp.int32),
            pltpu.VMEM((2, 128), jnp.int32),
            pltpu.VMEM((2, 128), jnp.float32),
            pltpu.SemaphoreType.DMA,
        ],
    )(ids_t, w_t, rank_t, base_sw.reshape(NW * E8 * L))


def _gather_body(RW, CH, T, rowtok_hbm, x_hbm, xs_hbm, idx_v, buf_v, sem):
    wid = lax.axis_index("s") * NC + lax.axis_index("c")
    base = wid * RW
    nch = RW // CH
    for c in range(nch):
        pltpu.sync_copy(rowtok_hbm.at[pl.ds(base + c * CH, CH)], idx_v.at[c])
    mask = jnp.full((L,), T - 1, jnp.int32)
    for c in range(nch):
        for q in range(CH // L):
            idx_v[c, pl.ds(q * L, L)] = idx_v[c, pl.ds(q * L, L)] & mask
    pltpu.async_copy(x_hbm.at[idx_v.at[0]], buf_v.at[0], sem)
    if nch > 1:
        pltpu.async_copy(x_hbm.at[idx_v.at[1]], buf_v.at[1], sem)
    for c in range(nch):
        pltpu.make_async_copy(
            x_hbm.at[idx_v.at[c]], buf_v.at[c % 2], sem).wait()
        pltpu.sync_copy(buf_v.at[c % 2], xs_hbm.at[pl.ds(base + c * CH, CH)])
        if c + 2 < nch:
            pltpu.async_copy(x_hbm.at[idx_v.at[c + 2]], buf_v.at[c % 2], sem)


def _run_gather(rowtok, flat, P, T, H):
    RW = P // NW
    CH = 32
    return pl.kernel(
        functools.partial(_gather_body, RW, CH, T),
        out_type=jax.ShapeDtypeStruct((P, H), jnp.float32),
        mesh=plsc.VectorSubcoreMesh(core_axis_name="c", subcore_axis_name="s"),
        scratch_types=[
            pltpu.VMEM((RW // CH, CH), jnp.int32),
            pltpu.VMEM((2, CH, H), jnp.float32),
            pltpu.SemaphoreType.DMA,
        ],
    )(rowtok, flat)


def _gmm1_body(g_ref, x_ref, w1_ref, b1_ref, h_ref):
    t = pl.program_id(0)

    @pl.when(g_ref[1, t] == 1)
    def _():
        x = x_ref[...].astype(jnp.bfloat16)
        w1 = w1_ref[0].astype(jnp.bfloat16)
        h = lax.dot_general(
            x, w1, (((1,), (1,)), ((), ())),
            preferred_element_type=jnp.float32) + b1_ref[0]
        h = 0.5 * h * (1.0 + lax.erf(h * jnp.float32(0.7071067811865476)))
        h_ref[...] = h.astype(jnp.bfloat16)


def _run_gmm1(group, xs, W1b, b1, P, E, F, H):
    NT = P // TM
    grid_spec = pltpu.PrefetchScalarGridSpec(
        num_scalar_prefetch=1,
        grid=(NT,),
        in_specs=[
            pl.BlockSpec((TM, H), lambda t, g: (t, 0)),
            pl.BlockSpec((1, F, H), lambda t, g: (g[0, t], 0, 0)),
            pl.BlockSpec((1, 1, F), lambda t, g: (g[0, t], 0, 0)),
        ],
        out_specs=pl.BlockSpec((TM, F), lambda t, g: (t, 0)),
    )
    return pl.pallas_call(
        _gmm1_body,
        grid_spec=grid_spec,
        out_shape=jax.ShapeDtypeStruct((P, F), jnp.bfloat16),
    )(group, xs, W1b, b1.reshape(E, 1, F))


def _gmm2s_body(g_ref, h_ref, w2_ref, b2_ref, w_ref, y_ref):
    t = pl.program_id(1)

    @pl.when(g_ref[1, t] == 1)
    def _():
        h = h_ref[...]
        w2 = w2_ref[0].astype(jnp.bfloat16)
        y = lax.dot_general(
            h, w2, (((1,), (1,)), ((), ())),
            preferred_element_type=jnp.float32) + b2_ref[0]
        y_ref[...] = y * w_ref[...]


def _run_gmm2s(group, h, W2, b2, roww, P, E, F, V):
    NT = P // TM
    VB = min(1024, V)
    NVB = V // VB
    grid_spec = pltpu.PrefetchScalarGridSpec(
        num_scalar_prefetch=1,
        grid=(NVB, NT),
        in_specs=[
            pl.BlockSpec((TM, F), lambda vb, t, g: (t, 0)),
            pl.BlockSpec((1, VB, F), lambda vb, t, g: (g[0, t], vb, 0)),
            pl.BlockSpec((1, 1, VB), lambda vb, t, g: (g[0, t], 0, vb)),
            pl.BlockSpec((TM, 1), lambda vb, t, g: (t, 0)),
        ],
        out_specs=pl.BlockSpec((TM, VB), lambda vb, t, g: (t, vb)),
    )
    return pl.pallas_call(
        _gmm2s_body,
        grid_spec=grid_spec,
        out_shape=jax.ShapeDtypeStruct((P, V), jnp.float32),
    )(group, h, W2, b2.reshape(E, 1, V), roww)


def _gmm2_body(g_ref, h_ref, w2_ref, b2_ref, w_ref, y_ref):
    t = pl.program_id(0)

    @pl.when(g_ref[1, t] == 1)
    def _():
        h = h_ref[...]
        y = lax.dot_general(
            h, w2_ref[0], (((1,), (1,)), ((), ())),
            preferred_element_type=jnp.float32) + b2_ref[0]
        y_ref[...] = y * w_ref[...]


def _run_gmm2(group, h, W2b, b2, roww, P, E, F, V):
    NT = P // TM
    grid_spec = pltpu.PrefetchScalarGridSpec(
        num_scalar_prefetch=1,
        grid=(NT,),
        in_specs=[
            pl.BlockSpec((TM, F), lambda t, g: (t, 0)),
            pl.BlockSpec((1, V, F), lambda t, g: (g[0, t], 0, 0)),
            pl.BlockSpec((1, 1, V), lambda t, g: (g[0, t], 0, 0)),
            pl.BlockSpec((TM, 1), lambda t, g: (t, 0)),
        ],
        out_specs=pl.BlockSpec((TM, V), lambda t, g: (t, 0)),
    )
    return pl.pallas_call(
        _gmm2_body,
        grid_spec=grid_spec,
        out_shape=jax.ShapeDtypeStruct((P, V), jnp.float32),
    )(group, h, W2b, b2.reshape(E, 1, V), roww)


def _combine_body(TW, V, y_hbm, pos_hbm, out_hbm, pos_v, ybuf_v, obuf_v, sem):
    wid = lax.axis_index("s") * NC + lax.axis_index("c")
    tbase = wid * TW
    ngrp = (2 * TW) // L
    for g in range(ngrp):
        pltpu.sync_copy(pos_hbm.at[pl.ds(2 * tbase + g * L, L)], pos_v.at[g])

    def body(g, _):
        pltpu.async_copy(y_hbm.at[pos_v.at[g]], ybuf_v.at[0], sem).wait()

        def inner(q, _):
            for i in range(L // 2):
                obuf_v[i, pl.ds(q * L, L)] = (
                    ybuf_v[0, 2 * i, pl.ds(q * L, L)]
                    + ybuf_v[0, 2 * i + 1, pl.ds(q * L, L)]
                )
            return 0

        lax.fori_loop(0, V // L, inner, 0)
        pltpu.sync_copy(obuf_v, out_hbm.at[pl.ds(tbase + g * (L // 2), L // 2)])
        return 0

    lax.fori_loop(0, ngrp, body, 0)


def _run_combine(y, pos, T, V):
    TW = T // NW
    return pl.kernel(
        functools.partial(_combine_body, TW, V),
        out_type=jax.ShapeDtypeStruct((T, V), jnp.float32),
        mesh=plsc.VectorSubcoreMesh(core_axis_name="c", subcore_axis_name="s"),
        scratch_types=[
            pltpu.VMEM(((2 * TW) // L, L), jnp.int32),
            pltpu.VMEM((2, L, V), jnp.float32),
            pltpu.VMEM((L // 2, V), jnp.float32),
            pltpu.SemaphoreType.DMA,
        ],
    )(y, pos)


def kernel(hidden_states, Wr, br, W1, b1, W2, b2):
    B, S, H = hidden_states.shape
    E, F, _ = W1.shape
    V = W2.shape[1]
    T = B * S
    T2 = 2 * T
    NT = T2 // TM + E
    NTPAD = ((NT + 15) // 16) * 16
    P = NT * TM

    flat = hidden_states.reshape(T, H)

    ids2, w2, _usage, loss11 = _run_router(flat, Wr, br, T, E, H)
    ids_t = ids2.reshape(NW, L, L).transpose(0, 2, 1).reshape(T2)
    w_t = w2.reshape(NW, L, L).transpose(0, 2, 1).reshape(T2)

    counts, rank_t = _run_disp1(ids_t, T2)
    base_sw, group2 = _run_prefix(counts, NTPAD)
    pos_t, rowtok, roww = _run_disp2(ids_t, w_t, rank_t, base_sw, T2, P)
    pos = pos_t.reshape(NW, L, L).transpose(0, 2, 1).reshape(T2)
    xs = _run_gather(rowtok, flat, P, T, H)
    group = group2[:, :NT]
    h = _run_gmm1(group, xs, W1, b1, P, E, F, H)
    y = _run_gmm2s(group, h, W2, b2, roww.reshape(P, 1), P, E, F, V)
    out = _run_combine(y, pos, T, V)

    return out.reshape(B, S, V), loss11[0, 0]
